# baseline (speedup 1.0000x reference)
import jax
import jax.numpy as jnp
from jax import lax
from jax.experimental import pallas as pl
from jax.experimental.pallas import tpu as pltpu
from jax.experimental.pallas import tpu_sc as plsc

B = 4096
D = 64
LEN_VOCAB = 51
OUT_W = 3 * D + LEN_VOCAB

_info = plsc.get_sparse_core_info()
NC = _info.num_cores
NS = _info.num_subcores
NW = NC * NS
BPW = B // NW

NSEM = 8

TC_BLK = 128
TC_GRID = B // TC_BLK


def _sc_gather_body(adv_id, brd_id, ind_id, adv_t, brd_t, ind_t,
                    g_adv, g_brd, g_ind, ia_s, ib_s, ii_s, *sems):
    wid = lax.axis_index("s") * NC + lax.axis_index("c")
    base = wid * BPW

    pltpu.sync_copy(adv_id.at[pl.ds(base, BPW)], ia_s)
    pltpu.sync_copy(brd_id.at[pl.ds(base, BPW)], ib_s)
    pltpu.sync_copy(ind_id.at[pl.ds(base, BPW)], ii_s)

    def issue(g, carry):
        gbase = g * 16
        av = ia_s[pl.ds(gbase, 16)]
        bv = ib_s[pl.ds(gbase, 16)]
        iv = ii_s[pl.ds(gbase, 16)]
        for j in range(16):
            row = base + gbase + j
            sem = sems[j % NSEM]
            pltpu.async_copy(adv_t.at[pl.ds(av[j], 1)],
                             g_adv.at[pl.ds(row, 1)], sem)
            pltpu.async_copy(brd_t.at[pl.ds(bv[j], 1)],
                             g_brd.at[pl.ds(row, 1)], sem)
            pltpu.async_copy(ind_t.at[pl.ds(iv[j], 1)],
                             g_ind.at[pl.ds(row, 1)], sem)
        return carry

    lax.fori_loop(0, BPW // 16, issue, 0)

    def drain(g, carry):
        for j in range(16):
            sem = sems[j % NSEM]
            pltpu.make_async_copy(adv_t.at[pl.ds(0, 1)],
                                  g_adv.at[pl.ds(0, 1)], sem).wait()
            pltpu.make_async_copy(brd_t.at[pl.ds(0, 1)],
                                  g_brd.at[pl.ds(0, 1)], sem).wait()
            pltpu.make_async_copy(ind_t.at[pl.ds(0, 1)],
                                  g_ind.at[pl.ds(0, 1)], sem).wait()
        return carry

    lax.fori_loop(0, BPW // 16, drain, 0)


def _tc_concat_body(adv_ref, brd_ref, ind_ref, cl_ref, out_ref):
    cl = cl_ref[0, 0, :]
    oh = (cl[:, None] == lax.broadcasted_iota(jnp.int32, (TC_BLK, LEN_VOCAB),
                                              1)).astype(jnp.float32)
    out_ref[...] = jnp.concatenate(
        [adv_ref[...], brd_ref[...], oh, ind_ref[...]], axis=1)


def kernel(advertiser_id, brand_id, industry, campaign_length,
           adv_table, brand_table, ind_table):
    mesh = plsc.VectorSubcoreMesh(core_axis_name="c", subcore_axis_name="s")
    gather = pl.kernel(
        _sc_gather_body,
        mesh=mesh,
        out_type=(
            jax.ShapeDtypeStruct((B, D), jnp.float32),
            jax.ShapeDtypeStruct((B, D), jnp.float32),
            jax.ShapeDtypeStruct((B, D), jnp.float32),
        ),
        scratch_types=[
            pltpu.VMEM((BPW,), jnp.int32),
            pltpu.VMEM((BPW,), jnp.int32),
            pltpu.VMEM((BPW,), jnp.int32),
        ] + [pltpu.SemaphoreType.DMA] * NSEM,
    )
    g_adv, g_brd, g_ind = gather(advertiser_id, brand_id, industry,
                                 adv_table, brand_table, ind_table)

    cl3 = campaign_length.reshape(TC_GRID, 1, TC_BLK)
    concat = pl.pallas_call(
        _tc_concat_body,
        grid=(TC_GRID,),
        in_specs=[
            pl.BlockSpec((TC_BLK, D), lambda i: (i, 0)),
            pl.BlockSpec((TC_BLK, D), lambda i: (i, 0)),
            pl.BlockSpec((TC_BLK, D), lambda i: (i, 0)),
            pl.BlockSpec((1, 1, TC_BLK), lambda i: (i, 0, 0)),
        ],
        out_specs=pl.BlockSpec((TC_BLK, OUT_W), lambda i: (i, 0)),
        out_shape=jax.ShapeDtypeStruct((B, OUT_W), jnp.float32),
    )
    return concat(g_adv, g_brd, g_ind, cl3)


# speedup vs baseline: 1.9921x; 1.9921x over previous
import jax
import jax.numpy as jnp
from jax import lax
from jax.experimental import pallas as pl
from jax.experimental.pallas import tpu as pltpu
from jax.experimental.pallas import tpu_sc as plsc

B = 4096
D = 64
LEN_VOCAB = 51
OUT_W = 3 * D + LEN_VOCAB
PAD_W = 256

_info = plsc.get_sparse_core_info()
NC = _info.num_cores
NS = _info.num_subcores
NW = NC * NS
BPW = B // NW


def _sc_body(adv_id, brd_id, ind_id, len_id, adv_t, brd_t, ind_t, eye_t,
             out, ia, ib, ii, il, ra, rb, ri, rl, out_v, sa, sb, si, sl):
    wid = lax.axis_index("s") * NC + lax.axis_index("c")
    base = wid * BPW

    pltpu.sync_copy(adv_id.at[pl.ds(base, BPW)], ia)
    pltpu.sync_copy(brd_id.at[pl.ds(base, BPW)], ib)
    pltpu.sync_copy(ind_id.at[pl.ds(base, BPW)], ii)
    pltpu.sync_copy(len_id.at[pl.ds(base, BPW)], il)

    ca = pltpu.async_copy(adv_t.at[ia], ra, sa)
    cb = pltpu.async_copy(brd_t.at[ib], rb, sb)
    ci = pltpu.async_copy(ind_t.at[ii], ri, si)
    cl = pltpu.async_copy(eye_t.at[il], rl, sl)
    ca.wait()
    cb.wait()
    ci.wait()
    cl.wait()

    def body(r, carry):
        for c in range(4):
            out_v[r, pl.ds(c * 16, 16)] = ra[r, pl.ds(c * 16, 16)]
        for c in range(4):
            out_v[r, pl.ds(D + c * 16, 16)] = rb[r, pl.ds(c * 16, 16)]
        for c in range(4):
            out_v[r, pl.ds(2 * D + c * 16, 16)] = rl[r, pl.ds(c * 16, 16)]
        for c in range(4):
            out_v[r, pl.ds(2 * D + LEN_VOCAB + c * 16, 16)] = ri[r, pl.ds(c * 16, 16)]
        return carry

    lax.fori_loop(0, BPW, body, 0)
    pltpu.sync_copy(out_v, out.at[pl.ds(base, BPW)])


def kernel(advertiser_id, brand_id, industry, campaign_length,
           adv_table, brand_table, ind_table):
    eye = jnp.eye(LEN_VOCAB, D, dtype=jnp.float32)
    mesh = plsc.VectorSubcoreMesh(core_axis_name="c", subcore_axis_name="s")
    f = pl.kernel(
        _sc_body,
        mesh=mesh,
        compiler_params=pltpu.CompilerParams(use_tc_tiling_on_sc=False),
        out_type=jax.ShapeDtypeStruct((B, PAD_W), jnp.float32),
        scratch_types=[
            pltpu.VMEM((BPW,), jnp.int32),
            pltpu.VMEM((BPW,), jnp.int32),
            pltpu.VMEM((BPW,), jnp.int32),
            pltpu.VMEM((BPW,), jnp.int32),
            pltpu.VMEM((BPW, D), jnp.float32),
            pltpu.VMEM((BPW, D), jnp.float32),
            pltpu.VMEM((BPW, D), jnp.float32),
            pltpu.VMEM((BPW, D), jnp.float32),
            pltpu.VMEM((BPW, PAD_W), jnp.float32),
            pltpu.SemaphoreType.DMA,
            pltpu.SemaphoreType.DMA,
            pltpu.SemaphoreType.DMA,
            pltpu.SemaphoreType.DMA,
        ],
    )
    padded = f(advertiser_id, brand_id, industry, campaign_length,
               adv_table, brand_table, ind_table, eye)
    return padded[:, :OUT_W]


# speedup vs baseline: 2.0468x; 1.0274x over previous
import jax
import jax.numpy as jnp
from jax import lax
from jax.experimental import pallas as pl
from jax.experimental.pallas import tpu as pltpu
from jax.experimental.pallas import tpu_sc as plsc

B = 4096
D = 64
LEN_VOCAB = 51
OUT_W = 3 * D + LEN_VOCAB
PAD_W = 256

_info = plsc.get_sparse_core_info()
NC = _info.num_cores
NS = _info.num_subcores
NW = NC * NS
BPW = B // NW


def _sc_body(adv_id, brd_id, ind_id, len_id, adv_t, brd_t, ind_t,
             out, ia, ib, ii, il, ra, rb, ri, out_v, sa, sb, si):
    wid = lax.axis_index("s") * NC + lax.axis_index("c")
    base = wid * BPW

    pltpu.sync_copy(adv_id.at[pl.ds(base, BPW)], ia)
    pltpu.sync_copy(brd_id.at[pl.ds(base, BPW)], ib)
    pltpu.sync_copy(ind_id.at[pl.ds(base, BPW)], ii)
    pltpu.sync_copy(len_id.at[pl.ds(base, BPW)], il)

    ca = pltpu.async_copy(adv_t.at[ia], ra, sa)
    cb = pltpu.async_copy(brd_t.at[ib], rb, sb)
    ci = pltpu.async_copy(ind_t.at[ii], ri, si)
    ca.wait()
    cb.wait()
    ci.wait()

    iota = lax.iota(jnp.int32, 16)
    one = jnp.full((16,), 1.0, jnp.float32)
    zero = jnp.zeros((16,), jnp.float32)

    def body(g, carry):
        s = g * 16
        vl = il[pl.ds(s, 16)]
        for j in range(16):
            r = s + j
            for c in range(4):
                out_v[r, pl.ds(c * 16, 16)] = ra[r, pl.ds(c * 16, 16)]
            for c in range(4):
                out_v[r, pl.ds(D + c * 16, 16)] = rb[r, pl.ds(c * 16, 16)]
            for c in range(4):
                out_v[r, pl.ds(2 * D + c * 16, 16)] = jnp.where(
                    iota + (c * 16) == vl[j], one, zero)
            for c in range(4):
                out_v[r, pl.ds(2 * D + LEN_VOCAB + c * 16, 16)] = \
                    ri[r, pl.ds(c * 16, 16)]
        return carry

    lax.fori_loop(0, BPW // 16, body, 0)
    pltpu.sync_copy(out_v, out.at[pl.ds(base, BPW)])


def kernel(advertiser_id, brand_id, industry, campaign_length,
           adv_table, brand_table, ind_table):
    mesh = plsc.VectorSubcoreMesh(core_axis_name="c", subcore_axis_name="s")
    f = pl.kernel(
        _sc_body,
        mesh=mesh,
        compiler_params=pltpu.CompilerParams(use_tc_tiling_on_sc=False),
        out_type=jax.ShapeDtypeStruct((B, PAD_W), jnp.float32),
        scratch_types=[
            pltpu.VMEM((BPW,), jnp.int32),
            pltpu.VMEM((BPW,), jnp.int32),
            pltpu.VMEM((BPW,), jnp.int32),
            pltpu.VMEM((BPW,), jnp.int32),
            pltpu.VMEM((BPW, D), jnp.float32),
            pltpu.VMEM((BPW, D), jnp.float32),
            pltpu.VMEM((BPW, D), jnp.float32),
            pltpu.VMEM((BPW, PAD_W), jnp.float32),
            pltpu.SemaphoreType.DMA,
            pltpu.SemaphoreType.DMA,
            pltpu.SemaphoreType.DMA,
        ],
    )
    padded = f(advertiser_id, brand_id, industry, campaign_length,
               adv_table, brand_table, ind_table)
    return padded[:, :OUT_W]
